# baseline (speedup 1.0000x reference)
import functools

import jax
import jax.numpy as jnp
from jax.experimental import pallas as pl
from jax.experimental.pallas import tpu as pltpu

_K = 32
_QT = 128
_C = 512
_BIG = 1e10


def _knn_body(off_ref, q_ref, r3_ref, out_ref, s_ref, *, nr, nb):
    cw = r3_ref.shape[2]
    q = q_ref[...]
    qt = q.shape[0]
    qb = q[:, 0:1]
    qx0, qx1, qx2 = q[:, 1:2], q[:, 2:3], q[:, 3:4]
    q2 = q[:, 4:5]

    def bf(x):
        return x.astype(jnp.bfloat16).astype(jnp.float32)

    qb0 = bf(qx0)
    qb1 = bf(qx1)
    qb2 = bf(qx2)

    b_lo = q_ref[0, 0].astype(jnp.int32)
    b_hi = q_ref[qt - 1, 0].astype(jnp.int32)
    start = off_ref[b_lo]
    end = off_ref[b_hi + 1]
    c0 = start // cw
    c1 = (end + cw - 1) // cw

    def compute_chunk(c, _):
        rt = r3_ref[c]
        rb = rt[0:1, :]
        rx0, rx1, rx2 = rt[1:2, :], rt[2:3, :], rt[3:4, :]
        r2 = rt[4:5, :]
        cross = qb0 * bf(rx0) + qb1 * bf(rx1) + qb2 * bf(rx2)
        d2 = q2 + r2 - 2.0 * cross
        s_ref[c] = jnp.where(qb == rb, d2, _BIG)
        return 0

    jax.lax.fori_loop(c0, c1, compute_chunk, 0)

    inf = float("inf")
    gr = min(32, qt)
    lane = jax.lax.broadcasted_iota(jnp.int32, (gr, cw), 1)
    for g in range(qt // gr):
        sl = slice(g * gr, (g + 1) * gr)
        gb_lo = q_ref[g * gr, 0].astype(jnp.int32)
        gb_hi = q_ref[(g + 1) * gr - 1, 0].astype(jnp.int32)
        gc0 = off_ref[gb_lo] // cw
        gc1 = (off_ref[gb_hi + 1] + cw - 1) // cw
        pi = jnp.full((gr, 1), -1, jnp.int32)
        for k in range(_K):
            g0 = gc0 * cw + lane
            s0 = jnp.where(g0 == pi, inf, s_ref[gc0, sl, :])
            s_ref[gc0, sl, :] = s0

            def chunk_body(c, carry):
                macc, iacc = carry
                gidx = c * cw + lane
                s = jnp.where(gidx == pi, inf, s_ref[c, sl, :])
                s_ref[c, sl, :] = s
                upd = s < macc
                return jnp.minimum(macc, s), jnp.where(upd, gidx, iacc)

            macc, iacc = jax.lax.fori_loop(gc0 + 1, gc1, chunk_body, (s0, g0))
            pv = jnp.min(macc, axis=1, keepdims=True)
            pi = jnp.min(jnp.where(macc == pv, iacc, nr), axis=1, keepdims=True)
            out_ref[sl, k : k + 1] = pi


def _knn_topk(query_s, ref_3d, offsets, *, interpret=False):
    nq = query_s.shape[0]
    nc, _, c = ref_3d.shape
    nr = nc * c
    nb = offsets.shape[0] - 1
    qt = min(_QT, nq)
    grid = (nq // qt,)
    return pl.pallas_call(
        functools.partial(_knn_body, nr=nr, nb=nb),
        grid=grid,
        in_specs=[
            pl.BlockSpec(memory_space=pltpu.SMEM),
            pl.BlockSpec((qt, 5), lambda i: (i, 0)),
            pl.BlockSpec((nc, 5, c), lambda i: (0, 0, 0)),
        ],
        out_specs=pl.BlockSpec((qt, _K), lambda i: (i, 0)),
        out_shape=jax.ShapeDtypeStruct((nq, _K), jnp.int32),
        scratch_shapes=[pltpu.VMEM((nc, qt, c), jnp.float32)],
        interpret=interpret,
    )(offsets, query_s, ref_3d)


def kernel(ref_bxyz, query_bxyz):
    nq = query_bxyz.shape[0]
    nr = ref_bxyz.shape[0]
    nb = 4
    rb = ref_bxyz[:, 0].astype(jnp.int32)
    qb = query_bxyz[:, 0].astype(jnp.int32)
    order_r = jnp.argsort(rb)
    order_q = jnp.argsort(qb)
    c = min(_C, nr)
    ref_s = ref_bxyz[order_r]
    r2 = jnp.sum(ref_s[:, 1:4] * ref_s[:, 1:4], axis=1)
    ref_3d = (jnp.concatenate([ref_s, r2[:, None]], axis=1)
              .T.reshape(5, nr // c, c).transpose(1, 0, 2))
    q_s = query_bxyz[order_q]
    q2 = jnp.sum(q_s[:, 1:4] * q_s[:, 1:4], axis=1)
    query_s = jnp.concatenate([q_s, q2[:, None]], axis=1)
    counts = jnp.bincount(rb, length=nb)
    offsets = jnp.concatenate(
        [jnp.zeros((1,), jnp.int32), jnp.cumsum(counts).astype(jnp.int32)])
    idx = _knn_topk(query_s, ref_3d, offsets)
    edge0 = order_r.astype(jnp.int64)[idx]
    edge1 = jnp.broadcast_to(order_q.astype(jnp.int64)[:, None], (nq, _K))
    return jnp.stack([edge0.reshape(-1), edge1.reshape(-1)], axis=0)


# speedup vs baseline: 1.3528x; 1.3528x over previous
import functools

import jax
import jax.numpy as jnp
from jax.experimental import pallas as pl
from jax.experimental.pallas import tpu as pltpu

_K = 32
_QT = 128
_C = 512
_BIG = 1e10


def _knn_body(off_ref, q_ref, r3_ref, out_ref, s_ref, m_ref, i_ref, *, nr, nb):
    cw = r3_ref.shape[2]
    q = q_ref[...]
    qt = q.shape[0]
    qb = q[:, 0:1]
    qx0, qx1, qx2 = q[:, 1:2], q[:, 2:3], q[:, 3:4]
    q2 = q[:, 4:5]

    def bf(x):
        return x.astype(jnp.bfloat16).astype(jnp.float32)

    qb0 = bf(qx0)
    qb1 = bf(qx1)
    qb2 = bf(qx2)

    b_lo = q_ref[0, 0].astype(jnp.int32)
    b_hi = q_ref[qt - 1, 0].astype(jnp.int32)
    start = off_ref[b_lo]
    end = off_ref[b_hi + 1]
    c0 = start // cw
    c1 = (end + cw - 1) // cw

    def compute_chunk(c, _):
        rt = r3_ref[c]
        rb = rt[0:1, :]
        rx0, rx1, rx2 = rt[1:2, :], rt[2:3, :], rt[3:4, :]
        r2 = rt[4:5, :]
        cross = qb0 * bf(rx0) + qb1 * bf(rx1) + qb2 * bf(rx2)
        d2 = q2 + r2 - 2.0 * cross
        s_ref[c] = jnp.where(qb == rb, d2, _BIG)
        return 0

    jax.lax.fori_loop((c0 // 2) * 2, ((c1 + 1) // 2) * 2, compute_chunk, 0)

    inf = float("inf")
    lane = jax.lax.broadcasted_iota(jnp.int32, (qt, cw), 1)
    c0e = (c0 // 2) * 2
    c1e = ((c1 + 1) // 2) * 2
    npair = (c1e - c0e) // 2

    def masked_pair(c, pi):
        gidx_a = c * cw + lane
        gidx_b = gidx_a + cw
        sa = jnp.where(gidx_a == pi, inf, s_ref[c])
        s_ref[c] = sa
        sb = jnp.where(gidx_b == pi, inf, s_ref[c + 1])
        s_ref[c + 1] = sb
        return jnp.minimum(sa, sb), jnp.where(sa <= sb, gidx_a, gidx_b)

    pi = jnp.full((qt, 1), -1, jnp.int32)
    for k in range(_K):
        pv0, pidx0 = masked_pair(c0e, pi)
        m_ref[...] = pv0
        i_ref[...] = pidx0

        def pair_body(j, prev):
            pv, pidx = masked_pair(c0e + 2 + 2 * j, prev)
            macc = m_ref[...]
            upd = pv < macc
            m_ref[...] = jnp.minimum(macc, pv)
            i_ref[...] = jnp.where(upd, pidx, i_ref[...])
            return prev

        jax.lax.fori_loop(0, npair - 1, pair_body, pi)
        macc = m_ref[...]
        pv = jnp.min(macc, axis=1, keepdims=True)
        pi = jnp.min(jnp.where(macc == pv, i_ref[...], nr), axis=1, keepdims=True)
        out_ref[:, k : k + 1] = pi


def _knn_topk(query_s, ref_3d, offsets, *, interpret=False):
    nq = query_s.shape[0]
    nc, _, c = ref_3d.shape
    nr = nc * c
    nb = offsets.shape[0] - 1
    qt = min(_QT, nq)
    grid = (nq // qt,)
    return pl.pallas_call(
        functools.partial(_knn_body, nr=nr, nb=nb),
        grid=grid,
        in_specs=[
            pl.BlockSpec(memory_space=pltpu.SMEM),
            pl.BlockSpec((qt, 5), lambda i: (i, 0)),
            pl.BlockSpec((nc, 5, c), lambda i: (0, 0, 0)),
        ],
        out_specs=pl.BlockSpec((qt, _K), lambda i: (i, 0)),
        out_shape=jax.ShapeDtypeStruct((nq, _K), jnp.int32),
        scratch_shapes=[pltpu.VMEM((nc, qt, c), jnp.float32),
                        pltpu.VMEM((qt, c), jnp.float32),
                        pltpu.VMEM((qt, c), jnp.int32)],
        interpret=interpret,
    )(offsets, query_s, ref_3d)


def kernel(ref_bxyz, query_bxyz):
    nq = query_bxyz.shape[0]
    nr = ref_bxyz.shape[0]
    nb = 4
    rb = ref_bxyz[:, 0].astype(jnp.int32)
    qb = query_bxyz[:, 0].astype(jnp.int32)
    order_r = jnp.argsort(rb)
    order_q = jnp.argsort(qb)
    c = min(_C, nr)
    ref_s = ref_bxyz[order_r]
    r2 = jnp.sum(ref_s[:, 1:4] * ref_s[:, 1:4], axis=1)
    ref_3d = (jnp.concatenate([ref_s, r2[:, None]], axis=1)
              .T.reshape(5, nr // c, c).transpose(1, 0, 2))
    q_s = query_bxyz[order_q]
    q2 = jnp.sum(q_s[:, 1:4] * q_s[:, 1:4], axis=1)
    query_s = jnp.concatenate([q_s, q2[:, None]], axis=1)
    counts = jnp.bincount(rb, length=nb)
    offsets = jnp.concatenate(
        [jnp.zeros((1,), jnp.int32), jnp.cumsum(counts).astype(jnp.int32)])
    idx = _knn_topk(query_s, ref_3d, offsets)
    edge0 = order_r.astype(jnp.int64)[idx]
    edge1 = jnp.broadcast_to(order_q.astype(jnp.int64)[:, None], (nq, _K))
    return jnp.stack([edge0.reshape(-1), edge1.reshape(-1)], axis=0)
